# Initial kernel scaffold; baseline (speedup 1.0000x reference)
import jax
import jax.numpy as jnp
from jax.experimental import pallas as pl


def kernel(node_vectors, node_features, We1, be1, We2, be2, Wi, bi, Wx1, bx1, Wxo, bxo, Wh1, bh1, Who, bho):
    raise NotImplementedError("write your pallas kernel here")


# speedup vs baseline: 14.7966x; 14.7966x over previous
import functools
import math

import jax
import jax.numpy as jnp
from jax.experimental import pallas as pl
from jax.experimental.pallas import tpu as pltpu

_BI = 16


def _egcl_kernel(n, bi,
                 x_blk_ref, xT_ref, x_full_ref, h_blk_ref, h_full_ref,
                 We1s_ref, We1r_ref, w0_ref, be1_ref,
                 We2_ref, be2_ref, wi_ref, bi_ref,
                 Wx1_ref, bx1_ref, wxo_ref, bxo_ref,
                 Wh1_ref, bh1_ref, Who_ref, bho_ref,
                 vec_out_ref, feat_out_ref):
    f32 = jnp.float32
    i0 = pl.program_id(0) * bi

    x_blk = x_blk_ref[...]
    xT = xT_ref[...]
    h_blk = h_blk_ref[...]

    n_row = jnp.sum(xT * xT, axis=0, keepdims=True)
    n_col = jnp.sum(x_blk * x_blk, axis=1, keepdims=True)
    G = jnp.dot(x_blk, xT, preferred_element_type=f32)
    len2 = jnp.maximum(n_col + n_row - 2.0 * G, 0.0) + 1e-16
    lengths = jnp.sqrt(len2)

    A_s = jnp.dot(h_full_ref[...], We1s_ref[...], preferred_element_type=f32)
    A_r = (jnp.dot(h_blk, We1r_ref[...], preferred_element_type=f32)
           + be1_ref[...])
    w0 = w0_ref[...]
    h1 = len2[:, :, None] * w0[None, :, :] + A_s[None, :, :] + A_r[:, None, :]
    h1 = h1 * jax.nn.sigmoid(h1)

    m = (jnp.dot(h1.reshape(bi * n, -1), We2_ref[...],
                 preferred_element_type=f32) + be2_ref[...])
    m3 = m.reshape(bi, n, -1)

    elog = jnp.sum(m3 * wi_ref[...][None, :, :], axis=2) + bi_ref[0, 0]
    rows = jax.lax.broadcasted_iota(jnp.int32, (bi, n), 0)
    cols = jax.lax.broadcasted_iota(jnp.int32, (bi, n), 1)
    e = jnp.where(cols == rows + i0, 0.0, jax.nn.sigmoid(elog))
    m_i = jnp.sum(m3 * e[:, :, None], axis=1)

    px1 = jnp.dot(m, Wx1_ref[...], preferred_element_type=f32) + bx1_ref[...]
    px1 = px1 * jax.nn.sigmoid(px1)
    px = (jnp.sum(px1.reshape(bi, n, -1) * wxo_ref[...][None, :, :], axis=2)
          + bxo_ref[0, 0])
    coef = px / (1.0 + lengths)
    csum = jnp.sum(coef, axis=1, keepdims=True)
    shift = csum * x_blk - jnp.dot(coef, x_full_ref[...],
                                   preferred_element_type=f32)
    vec_out_ref[...] = x_blk + shift * (1.0 / (n - 1))

    m_i_s = m_i * (1.0 / math.sqrt(n - 1.0))
    phin = jnp.concatenate([m_i_s, h_blk], axis=1)
    ph = jnp.dot(phin, Wh1_ref[...], preferred_element_type=f32) + bh1_ref[...]
    feat = jnp.dot(ph, Who_ref[...], preferred_element_type=f32) + bho_ref[...]
    feat_out_ref[...] = feat + h_blk


def kernel(node_vectors, node_features, We1, be1, We2, be2, Wi, bi,
           Wx1, bx1, Wxo, bxo, Wh1, bh1, Who, bho):
    n = node_vectors.shape[0]
    nvec = node_vectors.shape[1]
    h = node_features.shape[1]
    m = We1.shape[1]
    bi_blk = _BI

    x = node_vectors.reshape(n, 3)
    xT = x.T

    w0 = We1[0:1, :]
    We1s = We1[1:1 + h, :]
    We1r = We1[1 + h:, :]

    full = lambda a: pl.BlockSpec(a.shape, lambda i: (0,) * a.ndim)
    row2 = lambda a: pl.BlockSpec((1, a.shape[-1]), lambda i: (0, 0))

    grid = (n // bi_blk,)
    out_shapes = (
        jax.ShapeDtypeStruct((n, 3), jnp.float32),
        jax.ShapeDtypeStruct((n, h), jnp.float32),
    )
    in_specs = [
        pl.BlockSpec((bi_blk, 3), lambda i: (i, 0)),
        full(xT),
        full(x),
        pl.BlockSpec((bi_blk, h), lambda i: (i, 0)),
        full(node_features),
        full(We1s), full(We1r), row2(w0), row2(be1.reshape(1, m)),
        full(We2), row2(be2.reshape(1, h)),
        row2(Wi.reshape(1, h)), row2(bi.reshape(1, 1)),
        full(Wx1), row2(bx1.reshape(1, m)),
        row2(Wxo.reshape(1, m)), row2(bxo.reshape(1, 1)),
        full(Wh1), row2(bh1.reshape(1, m)),
        full(Who), row2(bho.reshape(1, h)),
    ]
    out_specs = (
        pl.BlockSpec((bi_blk, 3), lambda i: (i, 0)),
        pl.BlockSpec((bi_blk, h), lambda i: (i, 0)),
    )

    vec, feat = pl.pallas_call(
        functools.partial(_egcl_kernel, n, bi_blk),
        grid=grid,
        in_specs=in_specs,
        out_specs=out_specs,
        out_shape=out_shapes,
        compiler_params=pltpu.CompilerParams(
            dimension_semantics=("parallel",)),
    )(x, xT, x, node_features, node_features,
      We1s, We1r, w0, be1.reshape(1, m),
      We2, be2.reshape(1, h), Wi.reshape(1, h), bi.reshape(1, 1),
      Wx1, bx1.reshape(1, m), Wxo.reshape(1, m), bxo.reshape(1, 1),
      Wh1, bh1.reshape(1, m), Who, bho.reshape(1, h))

    return vec.reshape(n, nvec, 3), feat
